# baseline (speedup 1.0000x reference)
import functools

import jax
import jax.numpy as jnp
from jax import lax
from jax.experimental import pallas as pl
from jax.experimental.pallas import tpu as pltpu
from jax.experimental.pallas import tpu_sc as plsc

_ROWS = 128
_N = 32768
_L = 16
_NC = 2
_NS = 16
_NW = _NC * _NS
_RPW = _ROWS // _NW
_UNROLL = 8
_NEG = -3.0e38
_BISECT_ITERS = 32

_mesh = plsc.VectorSubcoreMesh(core_axis_name="c", subcore_axis_name="s")


@functools.partial(
    pl.kernel,
    out_type=jax.ShapeDtypeStruct((_ROWS, _N), jnp.float32),
    mesh=_mesh,
    scratch_types=[
        pltpu.VMEM((_N,), jnp.float32),
        pltpu.VMEM((_N,), jnp.float32),
        pltpu.VMEM((_N + _L,), jnp.float32),
        pltpu.SemaphoreType.DMA,
        pltpu.SemaphoreType.DMA,
        pltpu.SemaphoreType.DMA,
        pltpu.SemaphoreType.DMA,
    ],
    compiler_params=pltpu.CompilerParams(needs_layout_passes=False),
)
def _sc_sparsemax(x_hbm, o_hbm, row_a, row_b, cand_v,
                  sin_a, sin_b, sout_a, sout_b):
    wid = lax.axis_index("s") * _NC + lax.axis_index("c")
    base_row = wid * _RPW
    bufs = (row_a, row_b)
    sins = (sin_a, sin_b)
    souts = (sout_a, sout_b)

    h_in = [None] * _RPW
    h_out = [None] * _RPW
    h_in[0] = pltpu.async_copy(x_hbm.at[base_row], row_a, sin_a)

    for r in range(_RPW):
        p = r % 2
        row_v = bufs[p]
        with jax.named_scope("wait_in"):
            h_in[r].wait()

        def max_body(i, accs):
            base = i * (_L * _UNROLL)
            accs = list(accs)
            for u in range(_UNROLL):
                accs[u % 4] = jnp.maximum(
                    accs[u % 4], row_v[pl.ds(base + u * _L, _L)])
            return tuple(accs)

        with jax.named_scope("passA"):
            accs = plsc.parallel_loop(
                0, _N // (_L * _UNROLL), 1, unroll=2,
                carry=tuple(jnp.full((_L,), _NEG, jnp.float32)
                            for _ in range(4)),
            )(max_body)
        mrun = jnp.maximum(jnp.maximum(accs[0], accs[1]),
                           jnp.maximum(accs[2], accs[3]))
        m = jnp.max(mrun)
        lo = m - 1.0
        hi = m

        if r >= 1:
            with jax.named_scope("wait_out"):
                h_out[r - 1].wait()
        if r + 1 < _RPW:
            q = (r + 1) % 2
            h_in[r + 1] = pltpu.async_copy(
                x_hbm.at[base_row + r + 1], bufs[q], sins[q])

        iota = lax.iota(jnp.int32, _L)

        def cp_body(i, off_vec):
            base = i * (_L * _UNROLL)
            vs = [row_v[pl.ds(base + u * _L, _L)] for u in range(_UNROLL)]
            masks = [v > lo for v in vs]
            pcs = [plsc.all_reduce_population_count(mk) for mk in masks]
            quals = [pc > 0 for pc in pcs]
            vfs = [jnp.where(mk, v, _NEG) for mk, v in zip(masks, vs)]
            for u in range(_UNROLL):
                plsc.store_scatter(
                    cand_v, [off_vec + iota], vfs[u], mask=quals[u])
                off_vec = off_vec + jnp.where(quals[u], _L, 0)
            return off_vec

        with jax.named_scope("passB"):
            off_vec = plsc.parallel_loop(
                0, _N // (_L * _UNROLL), 1, unroll=1,
                carry=jnp.zeros((_L,), jnp.int32))(cp_body)
            off = off_vec[0]
        cand_v[pl.ds(off, _L)] = jnp.full((_L,), _NEG, jnp.float32)
        nv = (off + _L - 1) // _L

        def bis(_, carry):
            lo, hi = carry
            mid = 0.5 * (lo + hi)

            def fb(i, acc):
                v = cand_v[pl.ds(i * _L, _L)]
                return acc + jnp.maximum(v - mid, 0.0)

            acc = lax.fori_loop(0, nv, fb, jnp.zeros((_L,), jnp.float32))
            f = jnp.sum(acc)
            ge = f >= 1.0
            return jnp.where(ge, mid, lo), jnp.where(ge, hi, mid)

        with jax.named_scope("bisect"):
            lo, hi = lax.fori_loop(0, _BISECT_ITERS, bis, (lo, hi))
        tau = lo

        def ob(i, carry):
            base = i * (_L * _UNROLL)
            for u in range(_UNROLL):
                sl = pl.ds(base + u * _L, _L)
                row_v[sl] = jnp.maximum(row_v[sl] - tau, 0.0)
            return carry

        with jax.named_scope("passC"):
            plsc.parallel_loop(
                0, _N // (_L * _UNROLL), 1, unroll=1, carry=jnp.int32(0))(ob)
        h_out[r] = pltpu.async_copy(row_v, o_hbm.at[base_row + r], souts[p])

    h_out[_RPW - 1].wait()


@jax.jit
def kernel(logits):
    return _sc_sparsemax(logits.astype(jnp.float32))


# speedup vs baseline: 1.6424x; 1.6424x over previous
import functools

import jax
import jax.numpy as jnp
from jax import lax
from jax.experimental import pallas as pl
from jax.experimental.pallas import tpu as pltpu
from jax.experimental.pallas import tpu_sc as plsc

_ROWS = 128
_N = 32768
_L = 16
_NC = 2
_NS = 16
_NW = _NC * _NS
_RPW = _ROWS // _NW
_UNROLL = 8
_NEG = -3.0e38
_BISECT_ITERS = 26

_mesh = plsc.VectorSubcoreMesh(core_axis_name="c", subcore_axis_name="s")


@functools.partial(
    pl.kernel,
    out_type=jax.ShapeDtypeStruct((_ROWS, _N), jnp.float32),
    mesh=_mesh,
    scratch_types=[
        pltpu.VMEM((_N,), jnp.float32),
        pltpu.VMEM((_N,), jnp.float32),
        pltpu.VMEM((_N + _L,), jnp.float32),
        pltpu.SemaphoreType.DMA,
        pltpu.SemaphoreType.DMA,
        pltpu.SemaphoreType.DMA,
        pltpu.SemaphoreType.DMA,
    ],
    compiler_params=pltpu.CompilerParams(needs_layout_passes=False),
)
def _sc_sparsemax(x_hbm, o_hbm, row_a, row_b, cand_v,
                  sin_a, sin_b, sout_a, sout_b):
    wid = lax.axis_index("s") * _NC + lax.axis_index("c")
    base_row = wid * _RPW
    bufs = (row_a, row_b)
    sins = (sin_a, sin_b)
    souts = (sout_a, sout_b)

    h_in = [None] * _RPW
    h_out = [None] * _RPW
    h_in[0] = pltpu.async_copy(x_hbm.at[base_row], row_a, sin_a)

    for r in range(_RPW):
        p = r % 2
        row_v = bufs[p]
        h_in[r].wait()

        def max_body(i, accs):
            base = i * (_L * _UNROLL)
            accs = list(accs)
            for u in range(_UNROLL):
                accs[u % 4] = jnp.maximum(
                    accs[u % 4], row_v[pl.ds(base + u * _L, _L)])
            return tuple(accs)

        accs = plsc.parallel_loop(
            0, _N // (_L * _UNROLL), 1, unroll=2,
            carry=tuple(jnp.full((_L,), _NEG, jnp.float32) for _ in range(4)),
        )(max_body)
        mrun = jnp.maximum(jnp.maximum(accs[0], accs[1]),
                           jnp.maximum(accs[2], accs[3]))
        m = jnp.max(mrun)
        lo = m - 1.0
        hi = m

        if r >= 1:
            h_out[r - 1].wait()
        if r + 1 < _RPW:
            q = (r + 1) % 2
            h_in[r + 1] = pltpu.async_copy(
                x_hbm.at[base_row + r + 1], bufs[q], sins[q])

        iota = lax.iota(jnp.int32, _L)

        def cp_body(i, off_vec):
            base = i * (_L * _UNROLL)
            vs = [row_v[pl.ds(base + u * _L, _L)] for u in range(_UNROLL)]
            masks = [v > lo for v in vs]
            pcs = [plsc.all_reduce_population_count(mk) for mk in masks]
            quals = [pc > 0 for pc in pcs]
            vfs = [jnp.where(mk, v, _NEG) for mk, v in zip(masks, vs)]
            for u in range(_UNROLL):
                plsc.store_scatter(
                    cand_v, [off_vec + iota], vfs[u], mask=quals[u])
                off_vec = off_vec + jnp.where(quals[u], _L, 0)
            return off_vec

        off_vec = plsc.parallel_loop(
            0, _N // (_L * _UNROLL), 1, unroll=1,
            carry=jnp.zeros((_L,), jnp.int32))(cp_body)
        off1 = off_vec[0]
        nv1 = off1 // _L

        def xc_body(i, off):
            v = cand_v[pl.ds(i * _L, _L)]
            mask = v > lo
            pc = plsc.all_reduce_population_count(mask)
            plsc.store_compressed(cand_v.at[pl.ds(off, _L)], v, mask=mask)
            return off + pc[0]

        off = lax.fori_loop(0, nv1, xc_body, jnp.int32(0))
        cand_v[pl.ds(off, _L)] = jnp.full((_L,), _NEG, jnp.float32)
        nv = (off + _L - 1) // _L

        def bis(_, carry):
            lo, hi = carry
            mid = 0.5 * (lo + hi)

            def fb(i, acc):
                v = cand_v[pl.ds(i * _L, _L)]
                return acc + jnp.maximum(v - mid, 0.0)

            acc = lax.fori_loop(0, nv, fb, jnp.zeros((_L,), jnp.float32))
            f = jnp.sum(acc)
            ge = f >= 1.0
            return jnp.where(ge, mid, lo), jnp.where(ge, hi, mid)

        lo, hi = lax.fori_loop(0, _BISECT_ITERS, bis, (lo, hi))
        tau = lo

        def ob(i, carry):
            base = i * (_L * _UNROLL)
            for u in range(_UNROLL):
                sl = pl.ds(base + u * _L, _L)
                row_v[sl] = jnp.maximum(row_v[sl] - tau, 0.0)
            return carry

        plsc.parallel_loop(
            0, _N // (_L * _UNROLL), 1, unroll=1, carry=jnp.int32(0))(ob)
        h_out[r] = pltpu.async_copy(row_v, o_hbm.at[base_row + r], souts[p])

    h_out[_RPW - 1].wait()


@jax.jit
def kernel(logits):
    return _sc_sparsemax(logits.astype(jnp.float32))
